# baseline (speedup 1.0000x reference)
import functools

import jax
import jax.numpy as jnp
from jax import lax
from jax.experimental import pallas as pl
from jax.experimental.pallas import tpu as pltpu
from jax.experimental.pallas import tpu_sc as plsc

N_NODES = 10000
D = 128
CHUNK = 128
NC, NS = 2, 16
NW = NC * NS
N_SP = 10112
ROWS_PER_TILE = N_SP // NS
ZROWS = ROWS_PER_TILE
NBUF = 2
MM_BLOCK = 1000


def _mm_body(x_ref, w_ref, o_ref):
    o_ref[...] = jnp.dot(x_ref[...], w_ref[...], preferred_element_type=jnp.float32)


def _summ_body(a_ref, b_ref, w_ref, o_ref):
    o_ref[...] = jnp.dot(a_ref[...] + b_ref[...], w_ref[...],
                         preferred_element_type=jnp.float32)


def _add_body(a_ref, b_ref, o_ref):
    o_ref[...] = a_ref[...] + b_ref[...]


def _tc_matmul(x, w):
    grid = (N_NODES // MM_BLOCK,)
    return pl.pallas_call(
        _mm_body,
        grid=grid,
        in_specs=[
            pl.BlockSpec((MM_BLOCK, D), lambda i: (i, 0)),
            pl.BlockSpec((D, D), lambda i: (0, 0)),
        ],
        out_specs=pl.BlockSpec((MM_BLOCK, D), lambda i: (i, 0)),
        out_shape=jax.ShapeDtypeStruct((N_NODES, D), jnp.float32),
    )(x, w)


def _tc_sum_matmul(p, w):
    grid = (N_NODES // MM_BLOCK,)
    return pl.pallas_call(
        _summ_body,
        grid=grid,
        in_specs=[
            pl.BlockSpec((MM_BLOCK, D), lambda i: (i, 0)),
            pl.BlockSpec((MM_BLOCK, D), lambda i: (i, 0)),
            pl.BlockSpec((D, D), lambda i: (0, 0)),
        ],
        out_specs=pl.BlockSpec((MM_BLOCK, D), lambda i: (i, 0)),
        out_shape=jax.ShapeDtypeStruct((N_NODES, D), jnp.float32),
    )(p[0], p[1], w)


def _tc_sum(p):
    grid = (N_NODES // MM_BLOCK,)
    return pl.pallas_call(
        _add_body,
        grid=grid,
        in_specs=[
            pl.BlockSpec((MM_BLOCK, D), lambda i: (i, 0)),
            pl.BlockSpec((MM_BLOCK, D), lambda i: (i, 0)),
        ],
        out_specs=pl.BlockSpec((MM_BLOCK, D), lambda i: (i, 0)),
        out_shape=jax.ShapeDtypeStruct((N_NODES, D), jnp.float32),
    )(p[0], p[1])


def _make_sc_segsum(n_chunks):
    mesh = plsc.VectorSubcoreMesh(core_axis_name="c", subcore_axis_name="s")

    @functools.partial(
        pl.kernel,
        mesh=mesh,
        out_type=jax.ShapeDtypeStruct((NC, N_SP, D), jnp.float32),
        scratch_types=(
            [pltpu.VMEM((n_chunks, CHUNK), jnp.int32)]
            + [pltpu.VMEM((CHUNK, D), jnp.float32) for _ in range(NBUF)]
            + [pltpu.VMEM((CHUNK,), jnp.int32) for _ in range(2 * NBUF)]
            + [pltpu.VMEM_SHARED((N_SP, D), jnp.float32)]
            + [pltpu.SemaphoreType.DMA for _ in range(2 * NBUF)]
        ),
    )
    def segsum(h_hbm, pidx_hbm, out_hbm, pidx_v, *rest):
        rows = rest[:NBUF]
        sidx = rest[NBUF:2 * NBUF]
        didx = rest[2 * NBUF:3 * NBUF]
        acc_sh = rest[3 * NBUF]
        gsem = rest[3 * NBUF + 1:4 * NBUF + 1]
        ssem = rest[4 * NBUF + 1:]
        c = lax.axis_index("c")
        s = lax.axis_index("s")
        wid = s * NC + c

        pltpu.sync_copy(pidx_hbm.at[wid], pidx_v)

        z = jnp.zeros((16,), jnp.float32)

        def _zero_row(i, _):
            for k in range(D // 16):
                rows[0][i, pl.ds(k * 16, 16)] = z
            return 0

        lax.fori_loop(0, CHUNK, _zero_row, 0)
        zbase = s * ZROWS
        nfull = ZROWS // CHUNK
        for j in range(nfull):
            pltpu.sync_copy(rows[0], acc_sh.at[pl.ds(zbase + j * CHUNK, CHUNK)])
        rem = ZROWS - nfull * CHUNK
        if rem:
            pltpu.sync_copy(rows[0].at[pl.ds(0, rem)],
                            acc_sh.at[pl.ds(zbase + nfull * CHUNK, rem)])
        plsc.subcore_barrier()

        def unpack(j, b):
            for k in range(CHUNK // 16):
                v = pidx_v[j, pl.ds(k * 16, 16)]
                sidx[b][pl.ds(k * 16, 16)] = lax.bitwise_and(v, 0xFFFF)
                didx[b][pl.ds(k * 16, 16)] = lax.shift_right_logical(v, 16)

        def gst(j, b):
            pltpu.async_copy(h_hbm.at[sidx[b]], rows[b], gsem[b])

        def sst(j, b):
            pltpu.async_copy(rows[b], acc_sh.at[didx[b]], ssem[b], add=True)

        def gwait(b):
            pltpu.make_async_copy(h_hbm.at[sidx[b]], rows[b], gsem[b]).wait()

        def swait(b):
            pltpu.make_async_copy(h_hbm.at[sidx[b]], rows[b], ssem[b]).wait()

        for b in range(NBUF):
            unpack(b, b)
            gst(b, b)

        n_groups = n_chunks // NBUF

        @pl.loop(0, n_groups - 1)
        def _group(g):
            j0 = g * NBUF
            for b in range(NBUF):
                gwait(b)
                sst(j0 + b, b)
            for b in range(NBUF):
                swait(b)
                unpack(j0 + NBUF + b, b)
                gst(j0 + NBUF + b, b)

        for b in range(NBUF):
            gwait(b)
            sst(0, b)
        for b in range(NBUF):
            swait(b)
        plsc.subcore_barrier()

        pltpu.sync_copy(acc_sh.at[pl.ds(s * ROWS_PER_TILE, ROWS_PER_TILE)],
                        out_hbm.at[c, pl.ds(s * ROWS_PER_TILE, ROWS_PER_TILE)])

    return segsum


def kernel(x, edge_index, W1, W2, W3):
    src = jnp.asarray(edge_index[0], jnp.int32)
    dst = jnp.asarray(edge_index[1], jnp.int32)
    n_edges = src.shape[0]
    per_xfer = NW * CHUNK
    n_chunks = -(-n_edges // per_xfer)
    n_chunks = -(-n_chunks // NBUF) * NBUF
    e_pad = n_chunks * per_xfer
    pad = e_pad - n_edges
    if pad:
        src = jnp.concatenate([src, jnp.zeros((pad,), jnp.int32)])
        dst = jnp.concatenate([dst, jnp.full((pad,), N_NODES, jnp.int32)])
    pidx3 = jnp.bitwise_or(src, jnp.left_shift(dst, 16)).reshape(
        NW, n_chunks, CHUNK)

    segsum = _make_sc_segsum(n_chunks)

    h = _tc_matmul(x, W1)
    p = segsum(h, pidx3)
    h = _tc_sum_matmul(p, W2)
    p = segsum(h, pidx3)
    h = _tc_sum_matmul(p, W3)
    p = segsum(h, pidx3)
    return _tc_sum(p)


# speedup vs baseline: 3.2078x; 3.2078x over previous
import functools

import jax
import jax.numpy as jnp
from jax import lax
from jax.experimental import pallas as pl
from jax.experimental.pallas import tpu as pltpu
from jax.experimental.pallas import tpu_sc as plsc

N_NODES = 10000
D = 128
CHUNK = 128
NC, NS = 2, 16
NW = NC * NS
N_SP = 10112
ROWS_PER_TILE = N_SP // NS
ZROWS = ROWS_PER_TILE
NBUF = 2
MM_BLOCK = 1000


def _mm_body(x_ref, w_ref, o_ref):
    o_ref[...] = jnp.dot(x_ref[...], w_ref[...], preferred_element_type=jnp.float32)


def _summ_body(a_ref, b_ref, w_ref, o_ref):
    o_ref[...] = jnp.dot(a_ref[...] + b_ref[...], w_ref[...],
                         preferred_element_type=jnp.float32)


def _add_body(a_ref, b_ref, o_ref):
    o_ref[...] = a_ref[...] + b_ref[...]


def _tc_matmul(x, w):
    grid = (N_NODES // MM_BLOCK,)
    return pl.pallas_call(
        _mm_body,
        grid=grid,
        in_specs=[
            pl.BlockSpec((MM_BLOCK, D), lambda i: (i, 0)),
            pl.BlockSpec((D, D), lambda i: (0, 0)),
        ],
        out_specs=pl.BlockSpec((MM_BLOCK, D), lambda i: (i, 0)),
        out_shape=jax.ShapeDtypeStruct((N_NODES, D), jnp.float32),
    )(x, w)


def _tc_sum_matmul(p, w):
    grid = (N_NODES // MM_BLOCK,)
    return pl.pallas_call(
        _summ_body,
        grid=grid,
        in_specs=[
            pl.BlockSpec((MM_BLOCK, D), lambda i: (i, 0)),
            pl.BlockSpec((MM_BLOCK, D), lambda i: (i, 0)),
            pl.BlockSpec((D, D), lambda i: (0, 0)),
        ],
        out_specs=pl.BlockSpec((MM_BLOCK, D), lambda i: (i, 0)),
        out_shape=jax.ShapeDtypeStruct((N_NODES, D), jnp.float32),
    )(p[0], p[1], w)


def _tc_sum(p):
    grid = (N_NODES // MM_BLOCK,)
    return pl.pallas_call(
        _add_body,
        grid=grid,
        in_specs=[
            pl.BlockSpec((MM_BLOCK, D), lambda i: (i, 0)),
            pl.BlockSpec((MM_BLOCK, D), lambda i: (i, 0)),
        ],
        out_specs=pl.BlockSpec((MM_BLOCK, D), lambda i: (i, 0)),
        out_shape=jax.ShapeDtypeStruct((N_NODES, D), jnp.float32),
    )(p[0], p[1])


def _make_sc_segsum(n_chunks):
    mesh = plsc.VectorSubcoreMesh(core_axis_name="c", subcore_axis_name="s")

    @functools.partial(
        pl.kernel,
        mesh=mesh,
        out_type=jax.ShapeDtypeStruct((NC, N_SP, D), jnp.float32),
        scratch_types=(
            [pltpu.VMEM((n_chunks, CHUNK), jnp.int32)]
            + [pltpu.VMEM((CHUNK, D), jnp.float32) for _ in range(NBUF)]
            + [pltpu.VMEM((CHUNK,), jnp.int32) for _ in range(2 * NBUF)]
            + [pltpu.VMEM_SHARED((N_SP, D), jnp.float32)]
            + [pltpu.SemaphoreType.DMA for _ in range(2 * NBUF)]
        ),
    )
    def segsum(h_hbm, pidx_hbm, out_hbm, pidx_v, *rest):
        rows = rest[:NBUF]
        sidx = rest[NBUF:2 * NBUF]
        didx = rest[2 * NBUF:3 * NBUF]
        acc_sh = rest[3 * NBUF]
        gsem = rest[3 * NBUF + 1:4 * NBUF + 1]
        ssem = rest[4 * NBUF + 1:]
        c = lax.axis_index("c")
        s = lax.axis_index("s")
        wid = s * NC + c

        pltpu.sync_copy(pidx_hbm.at[wid], pidx_v)

        z = jnp.zeros((16,), jnp.float32)

        def _zero_row(i, _):
            for k in range(D // 16):
                rows[0][i, pl.ds(k * 16, 16)] = z
            return 0

        lax.fori_loop(0, CHUNK, _zero_row, 0)
        zbase = s * ZROWS
        nfull = ZROWS // CHUNK
        for j in range(nfull):
            pltpu.sync_copy(rows[0], acc_sh.at[pl.ds(zbase + j * CHUNK, CHUNK)])
        rem = ZROWS - nfull * CHUNK
        if rem:
            pltpu.sync_copy(rows[0].at[pl.ds(0, rem)],
                            acc_sh.at[pl.ds(zbase + nfull * CHUNK, rem)])
        plsc.subcore_barrier()

        def unpack(j, b):
            for k in range(CHUNK // 16):
                v = pidx_v[j, pl.ds(k * 16, 16)]
                sidx[b][pl.ds(k * 16, 16)] = lax.bitwise_and(v, 0xFFFF)
                didx[b][pl.ds(k * 16, 16)] = lax.shift_right_logical(v, 16)

        def gst(j, b):
            pltpu.async_copy(h_hbm.at[sidx[b]], rows[b], gsem[b])

        def sst(j, b):
            pltpu.async_copy(rows[b], acc_sh.at[didx[b]], ssem[b], add=True)

        def gwait(b):
            pltpu.make_async_copy(h_hbm.at[sidx[b]], rows[b], gsem[b]).wait()

        def swait(b):
            pltpu.make_async_copy(h_hbm.at[sidx[b]], rows[b], ssem[b]).wait()

        n = n_chunks
        unpack(0, 0)
        gst(0, 0)
        gwait(0); unpack(1, 1); gst(1, 1); sst(0, 0)
        gwait(1); swait(0); unpack(2, 0); gst(2, 0); sst(1, 1)

        @pl.loop(2, n - 2, step=2)
        def _grp(j0):
            gwait(0); swait(1); unpack(j0 + 1, 1); gst(j0 + 1, 1); sst(j0, 0)
            gwait(1); swait(0); unpack(j0 + 2, 0); gst(j0 + 2, 0); sst(j0 + 1, 1)

        gwait(0); swait(1); unpack(n - 1, 1); gst(n - 1, 1); sst(n - 2, 0)
        gwait(1); sst(n - 1, 1)
        swait(0)
        swait(1)
        plsc.subcore_barrier()

        pltpu.sync_copy(acc_sh.at[pl.ds(s * ROWS_PER_TILE, ROWS_PER_TILE)],
                        out_hbm.at[c, pl.ds(s * ROWS_PER_TILE, ROWS_PER_TILE)])

    return segsum


def kernel(x, edge_index, W1, W2, W3):
    src = jnp.asarray(edge_index[0], jnp.int32)
    dst = jnp.asarray(edge_index[1], jnp.int32)
    n_edges = src.shape[0]
    per_xfer = NW * CHUNK
    n_chunks = -(-n_edges // per_xfer)
    n_chunks = -(-n_chunks // NBUF) * NBUF
    e_pad = n_chunks * per_xfer
    pad = e_pad - n_edges
    if pad:
        r = jnp.arange(pad, dtype=jnp.int32)
        src = jnp.concatenate([src, (r * 131) % N_NODES])
        dst = jnp.concatenate([dst, N_NODES + (r % (N_SP - N_NODES))])
    pidx3 = jnp.bitwise_or(src, jnp.left_shift(dst, 16)).reshape(
        NW, n_chunks, CHUNK)

    segsum = _make_sc_segsum(n_chunks)

    h = _tc_matmul(x, W1)
    p = segsum(h, pidx3)
    h = _tc_sum_matmul(p, W2)
    p = segsum(h, pidx3)
    h = _tc_sum_matmul(p, W3)
    p = segsum(h, pidx3)
    return _tc_sum(p)


# speedup vs baseline: 3.8069x; 1.1868x over previous
import functools

import jax
import jax.numpy as jnp
from jax import lax
from jax.experimental import pallas as pl
from jax.experimental.pallas import tpu as pltpu
from jax.experimental.pallas import tpu_sc as plsc

N_NODES = 10000
D = 128
CHUNK = 128
NC, NS = 2, 16
NW = NC * NS
N_SP = 10112
ROWS_PER_TILE = N_SP // NS
ZROWS = ROWS_PER_TILE
NBUF = 2
MM_BLOCK = 1000


def _mm_body(x_ref, w_ref, o_ref):
    o_ref[...] = jnp.dot(x_ref[...], w_ref[...], preferred_element_type=jnp.float32)


def _summ_body(a_ref, b_ref, w_ref, o_ref):
    o_ref[...] = jnp.dot(a_ref[...] + b_ref[...], w_ref[...],
                         preferred_element_type=jnp.float32)


def _add_body(a_ref, b_ref, o_ref):
    o_ref[...] = a_ref[...] + b_ref[...]


def _tc_matmul(x, w):
    grid = (N_NODES // MM_BLOCK,)
    return pl.pallas_call(
        _mm_body,
        grid=grid,
        in_specs=[
            pl.BlockSpec((MM_BLOCK, D), lambda i: (i, 0)),
            pl.BlockSpec((D, D), lambda i: (0, 0)),
        ],
        out_specs=pl.BlockSpec((MM_BLOCK, D), lambda i: (i, 0)),
        out_shape=jax.ShapeDtypeStruct((N_NODES, D), jnp.float32),
    )(x, w)


def _tc_sum_matmul(p, w):
    grid = (N_NODES // MM_BLOCK,)
    return pl.pallas_call(
        _summ_body,
        grid=grid,
        in_specs=[
            pl.BlockSpec((MM_BLOCK, D), lambda i: (i, 0)),
            pl.BlockSpec((MM_BLOCK, D), lambda i: (i, 0)),
            pl.BlockSpec((D, D), lambda i: (0, 0)),
        ],
        out_specs=pl.BlockSpec((MM_BLOCK, D), lambda i: (i, 0)),
        out_shape=jax.ShapeDtypeStruct((N_NODES, D), jnp.float32),
    )(p[0], p[1], w)


def _tc_sum(p):
    grid = (N_NODES // MM_BLOCK,)
    return pl.pallas_call(
        _add_body,
        grid=grid,
        in_specs=[
            pl.BlockSpec((MM_BLOCK, D), lambda i: (i, 0)),
            pl.BlockSpec((MM_BLOCK, D), lambda i: (i, 0)),
        ],
        out_specs=pl.BlockSpec((MM_BLOCK, D), lambda i: (i, 0)),
        out_shape=jax.ShapeDtypeStruct((N_NODES, D), jnp.float32),
    )(p[0], p[1])


def _make_sc_segsum(n_chunks):
    mesh = plsc.VectorSubcoreMesh(core_axis_name="c", subcore_axis_name="s")

    @functools.partial(
        pl.kernel,
        mesh=mesh,
        out_type=jax.ShapeDtypeStruct((NC, N_SP, D), jnp.float32),
        scratch_types=(
            [pltpu.VMEM((n_chunks, CHUNK), jnp.int32)]
            + [pltpu.VMEM((CHUNK, D), jnp.float32) for _ in range(NBUF)]
            + [pltpu.VMEM((CHUNK,), jnp.int32) for _ in range(2 * NBUF)]
            + [pltpu.VMEM_SHARED((N_SP, D), jnp.float32)]
            + [pltpu.SemaphoreType.DMA for _ in range(2 * NBUF)]
        ),
    )
    def segsum(h_hbm, pidx_hbm, out_hbm, pidx_v, *rest):
        rows = rest[:NBUF]
        sidx = rest[NBUF:2 * NBUF]
        didx = rest[2 * NBUF:3 * NBUF]
        acc_sh = rest[3 * NBUF]
        gsem = rest[3 * NBUF + 1:4 * NBUF + 1]
        ssem = rest[4 * NBUF + 1:]
        c = lax.axis_index("c")
        s = lax.axis_index("s")
        wid = s * NC + c

        pltpu.sync_copy(pidx_hbm.at[wid], pidx_v)

        z = jnp.zeros((16,), jnp.float32)

        def _zero_row(i, _):
            for k in range(D // 16):
                rows[0][i, pl.ds(k * 16, 16)] = z
            return 0

        lax.fori_loop(0, CHUNK, _zero_row, 0)
        zbase = s * ZROWS
        nfull = ZROWS // CHUNK
        for j in range(nfull):
            pltpu.sync_copy(rows[0], acc_sh.at[pl.ds(zbase + j * CHUNK, CHUNK)])
        rem = ZROWS - nfull * CHUNK
        if rem:
            pltpu.sync_copy(rows[0].at[pl.ds(0, rem)],
                            acc_sh.at[pl.ds(zbase + nfull * CHUNK, rem)])
        plsc.subcore_barrier()

        def unpack(j, b):
            for k in range(CHUNK // 16):
                v = pidx_v[j, pl.ds(k * 16, 16)]
                sidx[b][pl.ds(k * 16, 16)] = lax.bitwise_and(v, 0xFFFF)
                didx[b][pl.ds(k * 16, 16)] = lax.shift_right_logical(v, 16)

        def gst(j, b):
            pltpu.async_copy(h_hbm.at[sidx[b]], rows[b], gsem[b])

        def sst(j, b):
            pltpu.async_copy(rows[b], acc_sh.at[didx[b]], ssem[b], add=True)

        def gwait(b):
            pltpu.make_async_copy(h_hbm.at[sidx[b]], rows[b], gsem[b]).wait()

        def swait(b):
            pltpu.make_async_copy(h_hbm.at[sidx[b]], rows[b], ssem[b]).wait()

        n = n_chunks
        unpack(0, 0); gst(0, 0)
        unpack(1, 1); gst(1, 1); gwait(0); sst(0, 0)

        @pl.loop(2, n, step=2)
        def _grp(j0):
            swait(0); unpack(j0, 0); gst(j0, 0); gwait(1); sst(j0 - 1, 1)
            swait(1); unpack(j0 + 1, 1); gst(j0 + 1, 1); gwait(0); sst(j0, 0)

        gwait(1); sst(n - 1, 1)
        swait(0)
        swait(1)
        plsc.subcore_barrier()

        pltpu.sync_copy(acc_sh.at[pl.ds(s * ROWS_PER_TILE, ROWS_PER_TILE)],
                        out_hbm.at[c, pl.ds(s * ROWS_PER_TILE, ROWS_PER_TILE)])

    return segsum


def kernel(x, edge_index, W1, W2, W3):
    src = jnp.asarray(edge_index[0], jnp.int32)
    dst = jnp.asarray(edge_index[1], jnp.int32)
    n_edges = src.shape[0]
    per_xfer = NW * CHUNK
    n_chunks = -(-n_edges // per_xfer)
    n_chunks = -(-n_chunks // NBUF) * NBUF
    e_pad = n_chunks * per_xfer
    pad = e_pad - n_edges
    if pad:
        r = jnp.arange(pad, dtype=jnp.int32)
        src = jnp.concatenate([src, (r * 131) % N_NODES])
        dst = jnp.concatenate([dst, N_NODES + (r % (N_SP - N_NODES))])
    pidx3 = jnp.bitwise_or(src, jnp.left_shift(dst, 16)).reshape(
        NW, n_chunks, CHUNK)

    segsum = _make_sc_segsum(n_chunks)

    h = _tc_matmul(x, W1)
    p = segsum(h, pidx3)
    h = _tc_sum_matmul(p, W2)
    p = segsum(h, pidx3)
    h = _tc_sum_matmul(p, W3)
    p = segsum(h, pidx3)
    return _tc_sum(p)


# speedup vs baseline: 4.2405x; 1.1139x over previous
import functools

import jax
import jax.numpy as jnp
from jax import lax
from jax.experimental import pallas as pl
from jax.experimental.pallas import tpu as pltpu
from jax.experimental.pallas import tpu_sc as plsc

N_NODES = 10000
D = 128
CHUNK = 64
NC, NS = 2, 16
NW = NC * NS
N_SP = 10112
ROWS_PER_TILE = N_SP // NS
ZROWS = ROWS_PER_TILE
NBUF = 4
MM_BLOCK = 1000


def _mm_body(x_ref, w_ref, o_ref):
    o_ref[...] = jnp.dot(x_ref[...], w_ref[...], preferred_element_type=jnp.float32)


def _summ_body(a_ref, b_ref, w_ref, o_ref):
    o_ref[...] = jnp.dot(a_ref[...] + b_ref[...], w_ref[...],
                         preferred_element_type=jnp.float32)


def _add_body(a_ref, b_ref, o_ref):
    o_ref[...] = a_ref[...] + b_ref[...]


def _tc_matmul(x, w):
    grid = (N_NODES // MM_BLOCK,)
    return pl.pallas_call(
        _mm_body,
        grid=grid,
        in_specs=[
            pl.BlockSpec((MM_BLOCK, D), lambda i: (i, 0)),
            pl.BlockSpec((D, D), lambda i: (0, 0)),
        ],
        out_specs=pl.BlockSpec((MM_BLOCK, D), lambda i: (i, 0)),
        out_shape=jax.ShapeDtypeStruct((N_NODES, D), jnp.float32),
    )(x, w)


def _tc_sum_matmul(p, w):
    grid = (N_NODES // MM_BLOCK,)
    return pl.pallas_call(
        _summ_body,
        grid=grid,
        in_specs=[
            pl.BlockSpec((MM_BLOCK, D), lambda i: (i, 0)),
            pl.BlockSpec((MM_BLOCK, D), lambda i: (i, 0)),
            pl.BlockSpec((D, D), lambda i: (0, 0)),
        ],
        out_specs=pl.BlockSpec((MM_BLOCK, D), lambda i: (i, 0)),
        out_shape=jax.ShapeDtypeStruct((N_NODES, D), jnp.float32),
    )(p[0], p[1], w)


def _tc_sum(p):
    grid = (N_NODES // MM_BLOCK,)
    return pl.pallas_call(
        _add_body,
        grid=grid,
        in_specs=[
            pl.BlockSpec((MM_BLOCK, D), lambda i: (i, 0)),
            pl.BlockSpec((MM_BLOCK, D), lambda i: (i, 0)),
        ],
        out_specs=pl.BlockSpec((MM_BLOCK, D), lambda i: (i, 0)),
        out_shape=jax.ShapeDtypeStruct((N_NODES, D), jnp.float32),
    )(p[0], p[1])


def _make_sc_segsum(n_chunks):
    mesh = plsc.VectorSubcoreMesh(core_axis_name="c", subcore_axis_name="s")

    @functools.partial(
        pl.kernel,
        mesh=mesh,
        out_type=jax.ShapeDtypeStruct((NC, N_SP, D), jnp.float32),
        scratch_types=(
            [pltpu.VMEM((n_chunks // 2, 2 * CHUNK), jnp.int32)]
            + [pltpu.VMEM((CHUNK, D), jnp.float32) for _ in range(NBUF)]
            + [pltpu.VMEM((CHUNK,), jnp.int32) for _ in range(2 * NBUF)]
            + [pltpu.VMEM_SHARED((N_SP, D), jnp.float32)]
            + [pltpu.SemaphoreType.DMA for _ in range(2 * NBUF)]
        ),
    )
    def segsum(h_hbm, pidx_hbm, out_hbm, pidx_v, *rest):
        rows = rest[:NBUF]
        sidx = rest[NBUF:2 * NBUF]
        didx = rest[2 * NBUF:3 * NBUF]
        acc_sh = rest[3 * NBUF]
        gsem = rest[3 * NBUF + 1:4 * NBUF + 1]
        ssem = rest[4 * NBUF + 1:]
        c = lax.axis_index("c")
        s = lax.axis_index("s")
        wid = s * NC + c

        pltpu.sync_copy(pidx_hbm.at[wid], pidx_v)

        z = jnp.zeros((16,), jnp.float32)

        def _zero_row(i, _):
            for k in range(D // 16):
                rows[0][i, pl.ds(k * 16, 16)] = z
            return 0

        lax.fori_loop(0, CHUNK, _zero_row, 0)
        zbase = s * ZROWS
        nfull = ZROWS // CHUNK
        for j in range(nfull):
            pltpu.sync_copy(rows[0], acc_sh.at[pl.ds(zbase + j * CHUNK, CHUNK)])
        rem = ZROWS - nfull * CHUNK
        if rem:
            pltpu.sync_copy(rows[0].at[pl.ds(0, rem)],
                            acc_sh.at[pl.ds(zbase + nfull * CHUNK, rem)])
        plsc.subcore_barrier()

        def unpack(row, col, b):
            for k in range(CHUNK // 16):
                v = pidx_v[row, pl.ds(col + k * 16, 16)]
                sidx[b][pl.ds(k * 16, 16)] = lax.bitwise_and(v, 0xFFFF)
                didx[b][pl.ds(k * 16, 16)] = lax.shift_right_logical(v, 16)

        def gst(b):
            pltpu.async_copy(h_hbm.at[sidx[b]], rows[b], gsem[b])

        def sst(b):
            pltpu.async_copy(rows[b], acc_sh.at[didx[b]], ssem[b], add=True)

        def gwait(b):
            pltpu.make_async_copy(h_hbm.at[sidx[b]], rows[b], gsem[b]).wait()

        def swait(b):
            pltpu.make_async_copy(h_hbm.at[sidx[b]], rows[b], ssem[b]).wait()

        n = n_chunks
        for j in range(3):
            unpack(j // 2, (j % 2) * CHUNK, j)
            gst(j)
        unpack(1, CHUNK, 3); gst(3); gwait(0); sst(0)

        @pl.loop(4, n, step=4)
        def _grp(j0):
            h = j0 // 2
            swait(0); unpack(h, 0, 0); gst(0); gwait(1); sst(1)
            swait(1); unpack(h, CHUNK, 1); gst(1); gwait(2); sst(2)
            swait(2); unpack(h + 1, 0, 2); gst(2); gwait(3); sst(3)
            swait(3); unpack(h + 1, CHUNK, 3); gst(3); gwait(0); sst(0)

        gwait(1); sst(1)
        gwait(2); sst(2)
        gwait(3); sst(3)
        for b in range(4):
            swait(b)
        plsc.subcore_barrier()

        pltpu.sync_copy(acc_sh.at[pl.ds(s * ROWS_PER_TILE, ROWS_PER_TILE)],
                        out_hbm.at[c, pl.ds(s * ROWS_PER_TILE, ROWS_PER_TILE)])

    return segsum


def kernel(x, edge_index, W1, W2, W3):
    src = jnp.asarray(edge_index[0], jnp.int32)
    dst = jnp.asarray(edge_index[1], jnp.int32)
    n_edges = src.shape[0]
    per_xfer = NW * CHUNK
    n_chunks = -(-n_edges // per_xfer)
    n_chunks = -(-n_chunks // NBUF) * NBUF
    e_pad = n_chunks * per_xfer
    pad = e_pad - n_edges
    if pad:
        r = jnp.arange(pad, dtype=jnp.int32)
        src = jnp.concatenate([src, (r * 131) % N_NODES])
        dst = jnp.concatenate([dst, N_NODES + (r % (N_SP - N_NODES))])
    pidx3 = jnp.bitwise_or(src, jnp.left_shift(dst, 16)).reshape(
        NW, n_chunks // 2, 2 * CHUNK)

    segsum = _make_sc_segsum(n_chunks)

    h = _tc_matmul(x, W1)
    p = segsum(h, pidx3)
    h = _tc_sum_matmul(p, W2)
    p = segsum(h, pidx3)
    h = _tc_sum_matmul(p, W3)
    p = segsum(h, pidx3)
    return _tc_sum(p)


# speedup vs baseline: 4.2448x; 1.0010x over previous
import functools

import jax
import jax.numpy as jnp
from jax import lax
from jax.experimental import pallas as pl
from jax.experimental.pallas import tpu as pltpu
from jax.experimental.pallas import tpu_sc as plsc

N_NODES = 10000
D = 128
CHUNK = 32
NC, NS = 2, 16
NW = NC * NS
N_SP = 10112
ROWS_PER_TILE = N_SP // NS
ZROWS = ROWS_PER_TILE
NBUF = 8
MM_BLOCK = 1000


def _mm_body(x_ref, w_ref, o_ref):
    o_ref[...] = jnp.dot(x_ref[...], w_ref[...], preferred_element_type=jnp.float32)


def _summ_body(a_ref, b_ref, w_ref, o_ref):
    o_ref[...] = jnp.dot(a_ref[...] + b_ref[...], w_ref[...],
                         preferred_element_type=jnp.float32)


def _add_body(a_ref, b_ref, o_ref):
    o_ref[...] = a_ref[...] + b_ref[...]


def _tc_matmul(x, w):
    grid = (N_NODES // MM_BLOCK,)
    return pl.pallas_call(
        _mm_body,
        grid=grid,
        in_specs=[
            pl.BlockSpec((MM_BLOCK, D), lambda i: (i, 0)),
            pl.BlockSpec((D, D), lambda i: (0, 0)),
        ],
        out_specs=pl.BlockSpec((MM_BLOCK, D), lambda i: (i, 0)),
        out_shape=jax.ShapeDtypeStruct((N_NODES, D), jnp.float32),
    )(x, w)


def _tc_sum_matmul(p, w):
    grid = (N_NODES // MM_BLOCK,)
    return pl.pallas_call(
        _summ_body,
        grid=grid,
        in_specs=[
            pl.BlockSpec((MM_BLOCK, D), lambda i: (i, 0)),
            pl.BlockSpec((MM_BLOCK, D), lambda i: (i, 0)),
            pl.BlockSpec((D, D), lambda i: (0, 0)),
        ],
        out_specs=pl.BlockSpec((MM_BLOCK, D), lambda i: (i, 0)),
        out_shape=jax.ShapeDtypeStruct((N_NODES, D), jnp.float32),
    )(p[0], p[1], w)


def _tc_sum(p):
    grid = (N_NODES // MM_BLOCK,)
    return pl.pallas_call(
        _add_body,
        grid=grid,
        in_specs=[
            pl.BlockSpec((MM_BLOCK, D), lambda i: (i, 0)),
            pl.BlockSpec((MM_BLOCK, D), lambda i: (i, 0)),
        ],
        out_specs=pl.BlockSpec((MM_BLOCK, D), lambda i: (i, 0)),
        out_shape=jax.ShapeDtypeStruct((N_NODES, D), jnp.float32),
    )(p[0], p[1])


def _make_sc_segsum(n_chunks):
    mesh = plsc.VectorSubcoreMesh(core_axis_name="c", subcore_axis_name="s")

    @functools.partial(
        pl.kernel,
        mesh=mesh,
        out_type=jax.ShapeDtypeStruct((NC, N_SP, D), jnp.float32),
        scratch_types=(
            [pltpu.VMEM((n_chunks // 4, 4 * CHUNK), jnp.int32)]
            + [pltpu.VMEM((CHUNK, D), jnp.float32) for _ in range(NBUF)]
            + [pltpu.VMEM((CHUNK,), jnp.int32) for _ in range(2 * NBUF)]
            + [pltpu.VMEM_SHARED((N_SP, D), jnp.float32)]
            + [pltpu.SemaphoreType.DMA for _ in range(2 * NBUF)]
        ),
    )
    def segsum(h_hbm, pidx_hbm, out_hbm, pidx_v, *rest):
        rows = rest[:NBUF]
        sidx = rest[NBUF:2 * NBUF]
        didx = rest[2 * NBUF:3 * NBUF]
        acc_sh = rest[3 * NBUF]
        gsem = rest[3 * NBUF + 1:4 * NBUF + 1]
        ssem = rest[4 * NBUF + 1:]
        c = lax.axis_index("c")
        s = lax.axis_index("s")
        wid = s * NC + c

        pltpu.sync_copy(pidx_hbm.at[wid], pidx_v)

        z = jnp.zeros((16,), jnp.float32)

        def _zero_row(i, _):
            for k in range(D // 16):
                rows[0][i, pl.ds(k * 16, 16)] = z
            return 0

        lax.fori_loop(0, CHUNK, _zero_row, 0)
        zbase = s * ZROWS
        nfull = ZROWS // CHUNK
        for j in range(nfull):
            pltpu.sync_copy(rows[0], acc_sh.at[pl.ds(zbase + j * CHUNK, CHUNK)])
        rem = ZROWS - nfull * CHUNK
        if rem:
            pltpu.sync_copy(rows[0].at[pl.ds(0, rem)],
                            acc_sh.at[pl.ds(zbase + nfull * CHUNK, rem)])
        plsc.subcore_barrier()

        def unpack(row, col, b):
            for k in range(CHUNK // 16):
                v = pidx_v[row, pl.ds(col + k * 16, 16)]
                sidx[b][pl.ds(k * 16, 16)] = lax.bitwise_and(v, 0xFFFF)
                didx[b][pl.ds(k * 16, 16)] = lax.shift_right_logical(v, 16)

        def gst(b):
            pltpu.async_copy(h_hbm.at[sidx[b]], rows[b], gsem[b])

        def sst(b):
            pltpu.async_copy(rows[b], acc_sh.at[didx[b]], ssem[b], add=True)

        def gwait(b):
            pltpu.make_async_copy(h_hbm.at[sidx[b]], rows[b], gsem[b]).wait()

        def swait(b):
            pltpu.make_async_copy(h_hbm.at[sidx[b]], rows[b], ssem[b]).wait()

        n = n_chunks
        for j in range(NBUF - 1):
            unpack(j // 4, (j % 4) * CHUNK, j)
            gst(j)
        unpack((NBUF - 1) // 4, ((NBUF - 1) % 4) * CHUNK, NBUF - 1)
        gst(NBUF - 1); gwait(0); sst(0)

        @pl.loop(NBUF, n, step=NBUF)
        def _grp(j0):
            h = j0 // 4
            for b in range(NBUF):
                swait(b)
                unpack(h + b // 4, (b % 4) * CHUNK, b)
                gst(b)
                gwait((b + 1) % NBUF)
                sst((b + 1) % NBUF)

        for b in range(1, NBUF):
            gwait(b)
            sst(b)
        for b in range(NBUF):
            swait(b)
        plsc.subcore_barrier()

        pltpu.sync_copy(acc_sh.at[pl.ds(s * ROWS_PER_TILE, ROWS_PER_TILE)],
                        out_hbm.at[c, pl.ds(s * ROWS_PER_TILE, ROWS_PER_TILE)])

    return segsum


def kernel(x, edge_index, W1, W2, W3):
    src = jnp.asarray(edge_index[0], jnp.int32)
    dst = jnp.asarray(edge_index[1], jnp.int32)
    n_edges = src.shape[0]
    per_xfer = NW * CHUNK
    n_chunks = -(-n_edges // per_xfer)
    n_chunks = -(-n_chunks // NBUF) * NBUF
    e_pad = n_chunks * per_xfer
    pad = e_pad - n_edges
    if pad:
        r = jnp.arange(pad, dtype=jnp.int32)
        src = jnp.concatenate([src, (r * 131) % N_NODES])
        dst = jnp.concatenate([dst, N_NODES + (r % (N_SP - N_NODES))])
    pidx3 = jnp.bitwise_or(src, jnp.left_shift(dst, 16)).reshape(
        NW, n_chunks // 4, 4 * CHUNK)

    segsum = _make_sc_segsum(n_chunks)

    h = _tc_matmul(x, W1)
    p = segsum(h, pidx3)
    h = _tc_sum_matmul(p, W2)
    p = segsum(h, pidx3)
    h = _tc_sum_matmul(p, W3)
    p = segsum(h, pidx3)
    return _tc_sum(p)


# speedup vs baseline: 4.3055x; 1.0143x over previous
import functools

import jax
import jax.numpy as jnp
from jax import lax
from jax.experimental import pallas as pl
from jax.experimental.pallas import tpu as pltpu
from jax.experimental.pallas import tpu_sc as plsc

N_NODES = 10000
D = 128
CHUNK = 32
NC, NS = 2, 16
NW = NC * NS
N_SP = 10112
ROWS_PER_TILE = N_SP // NS
ZROWS = ROWS_PER_TILE
NBUF = 8
MM_BLOCK = 1000


def _mm_body(x_ref, w_ref, o_ref):
    o_ref[...] = jnp.dot(x_ref[...], w_ref[...], preferred_element_type=jnp.float32)


def _summ_body(a_ref, b_ref, w_ref, o_ref):
    o_ref[...] = jnp.dot(a_ref[...] + b_ref[...], w_ref[...],
                         preferred_element_type=jnp.float32)


def _add_body(a_ref, b_ref, o_ref):
    o_ref[...] = a_ref[...] + b_ref[...]


def _tc_matmul(x, w):
    grid = (N_NODES // MM_BLOCK,)
    return pl.pallas_call(
        _mm_body,
        grid=grid,
        in_specs=[
            pl.BlockSpec((MM_BLOCK, D), lambda i: (i, 0)),
            pl.BlockSpec((D, D), lambda i: (0, 0)),
        ],
        out_specs=pl.BlockSpec((MM_BLOCK, D), lambda i: (i, 0)),
        out_shape=jax.ShapeDtypeStruct((N_NODES, D), jnp.float32),
    )(x, w)


def _tc_sum_matmul(p, w):
    grid = (N_NODES // MM_BLOCK,)
    return pl.pallas_call(
        _summ_body,
        grid=grid,
        in_specs=[
            pl.BlockSpec((MM_BLOCK, D), lambda i: (i, 0)),
            pl.BlockSpec((MM_BLOCK, D), lambda i: (i, 0)),
            pl.BlockSpec((D, D), lambda i: (0, 0)),
        ],
        out_specs=pl.BlockSpec((MM_BLOCK, D), lambda i: (i, 0)),
        out_shape=jax.ShapeDtypeStruct((N_NODES, D), jnp.float32),
    )(p[0], p[1], w)


def _tc_sum(p):
    grid = (N_NODES // MM_BLOCK,)
    return pl.pallas_call(
        _add_body,
        grid=grid,
        in_specs=[
            pl.BlockSpec((MM_BLOCK, D), lambda i: (i, 0)),
            pl.BlockSpec((MM_BLOCK, D), lambda i: (i, 0)),
        ],
        out_specs=pl.BlockSpec((MM_BLOCK, D), lambda i: (i, 0)),
        out_shape=jax.ShapeDtypeStruct((N_NODES, D), jnp.float32),
    )(p[0], p[1])


def _make_sc_segsum(n_chunks):
    mesh = plsc.VectorSubcoreMesh(core_axis_name="c", subcore_axis_name="s")

    @functools.partial(
        pl.kernel,
        mesh=mesh,
        out_type=jax.ShapeDtypeStruct((NC, N_SP, D), jnp.float32),
        scratch_types=(
            [pltpu.VMEM((n_chunks // 4, 4 * CHUNK), jnp.int32)]
            + [pltpu.VMEM((CHUNK, D), jnp.float32) for _ in range(NBUF)]
            + [pltpu.VMEM((CHUNK,), jnp.int32) for _ in range(2 * NBUF)]
            + [pltpu.VMEM_SHARED((N_SP, D), jnp.float32)]
            + [pltpu.SemaphoreType.DMA for _ in range(2 * NBUF)]
        ),
    )
    def segsum(h_hbm, pidx_hbm, out_hbm, pidx_v, *rest):
        rows = rest[:NBUF]
        sidx = rest[NBUF:2 * NBUF]
        didx = rest[2 * NBUF:3 * NBUF]
        acc_sh = rest[3 * NBUF]
        gsem = rest[3 * NBUF + 1:4 * NBUF + 1]
        ssem = rest[4 * NBUF + 1:]
        c = lax.axis_index("c")
        s = lax.axis_index("s")
        wid = s * NC + c

        pltpu.sync_copy(pidx_hbm.at[wid], pidx_v)

        def unpack(row, col, b):
            for k in range(CHUNK // 16):
                v = pidx_v[row, pl.ds(col + k * 16, 16)]
                sidx[b][pl.ds(k * 16, 16)] = lax.bitwise_and(v, 0xFFFF)
                didx[b][pl.ds(k * 16, 16)] = lax.shift_right_logical(v, 16)

        def gst(b):
            pltpu.async_copy(h_hbm.at[sidx[b]], rows[b], gsem[b])

        def sst(b):
            pltpu.async_copy(rows[b], acc_sh.at[didx[b]], ssem[b], add=True)

        def gwait(b):
            pltpu.make_async_copy(h_hbm.at[sidx[b]], rows[b], gsem[b]).wait()

        def swait(b):
            pltpu.make_async_copy(h_hbm.at[sidx[b]], rows[b], ssem[b]).wait()

        n = n_chunks
        for j in range(NBUF - 1):
            unpack(j // 4, (j % 4) * CHUNK, j)
            gst(j)

        z = jnp.zeros((16,), jnp.float32)
        zb = rows[NBUF - 1]

        def _zero_row(i, _):
            for k in range(D // 16):
                zb[i, pl.ds(k * 16, 16)] = z
            return 0

        lax.fori_loop(0, CHUNK, _zero_row, 0)
        zbase = s * ZROWS
        nfull = ZROWS // CHUNK
        for j in range(nfull):
            pltpu.sync_copy(zb, acc_sh.at[pl.ds(zbase + j * CHUNK, CHUNK)])
        rem = ZROWS - nfull * CHUNK
        if rem:
            pltpu.sync_copy(zb.at[pl.ds(0, rem)],
                            acc_sh.at[pl.ds(zbase + nfull * CHUNK, rem)])
        plsc.subcore_barrier()

        unpack((NBUF - 1) // 4, ((NBUF - 1) % 4) * CHUNK, NBUF - 1)
        gst(NBUF - 1); gwait(0); sst(0)

        @pl.loop(NBUF, n, step=NBUF)
        def _grp(j0):
            h = j0 // 4
            for b in range(NBUF):
                swait(b)
                unpack(h + b // 4, (b % 4) * CHUNK, b)
                gst(b)
                gwait((b + 1) % NBUF)
                sst((b + 1) % NBUF)

        for b in range(1, NBUF):
            gwait(b)
            sst(b)
        for b in range(NBUF):
            swait(b)
        plsc.subcore_barrier()

        pltpu.sync_copy(acc_sh.at[pl.ds(s * ROWS_PER_TILE, ROWS_PER_TILE)],
                        out_hbm.at[c, pl.ds(s * ROWS_PER_TILE, ROWS_PER_TILE)])

    return segsum


def kernel(x, edge_index, W1, W2, W3):
    src = jnp.asarray(edge_index[0], jnp.int32)
    dst = jnp.asarray(edge_index[1], jnp.int32)
    n_edges = src.shape[0]
    per_xfer = NW * CHUNK
    n_chunks = -(-n_edges // per_xfer)
    n_chunks = -(-n_chunks // NBUF) * NBUF
    e_pad = n_chunks * per_xfer
    pad = e_pad - n_edges
    if pad:
        r = jnp.arange(pad, dtype=jnp.int32)
        src = jnp.concatenate([src, (r * 131) % N_NODES])
        dst = jnp.concatenate([dst, N_NODES + (r % (N_SP - N_NODES))])
    pidx3 = jnp.bitwise_or(src, jnp.left_shift(dst, 16)).reshape(
        NW, n_chunks // 4, 4 * CHUNK)

    segsum = _make_sc_segsum(n_chunks)

    h = _tc_matmul(x, W1)
    p = segsum(h, pidx3)
    h = _tc_sum_matmul(p, W2)
    p = segsum(h, pidx3)
    h = _tc_sum_matmul(p, W3)
    p = segsum(h, pidx3)
    return _tc_sum(p)


# speedup vs baseline: 4.4289x; 1.0287x over previous
import functools

import jax
import jax.numpy as jnp
from jax import lax
from jax.experimental import pallas as pl
from jax.experimental.pallas import tpu as pltpu
from jax.experimental.pallas import tpu_sc as plsc

N_NODES = 10000
D = 128
CHUNK = 32
NC, NS = 2, 16
NW = NC * NS
N_SP = 10112
ROWS_PER_TILE = N_SP // NS
ZROWS = ROWS_PER_TILE
NBUF = 8
MM_BLOCK = 2000


def _mm_body(x_ref, w_ref, o_ref):
    o_ref[...] = jnp.dot(x_ref[...], w_ref[...], preferred_element_type=jnp.float32)


def _summ_body(a_ref, b_ref, w_ref, o_ref):
    o_ref[...] = jnp.dot(a_ref[...] + b_ref[...], w_ref[...],
                         preferred_element_type=jnp.float32)


def _add_body(a_ref, b_ref, o_ref):
    o_ref[...] = a_ref[...] + b_ref[...]


def _tc_matmul(x, w):
    grid = (N_NODES // MM_BLOCK,)
    return pl.pallas_call(
        _mm_body,
        grid=grid,
        in_specs=[
            pl.BlockSpec((MM_BLOCK, D), lambda i: (i, 0)),
            pl.BlockSpec((D, D), lambda i: (0, 0)),
        ],
        out_specs=pl.BlockSpec((MM_BLOCK, D), lambda i: (i, 0)),
        out_shape=jax.ShapeDtypeStruct((N_NODES, D), jnp.float32),
    )(x, w)


def _tc_sum_matmul(p, w):
    grid = (N_NODES // MM_BLOCK,)
    return pl.pallas_call(
        _summ_body,
        grid=grid,
        in_specs=[
            pl.BlockSpec((MM_BLOCK, D), lambda i: (i, 0)),
            pl.BlockSpec((MM_BLOCK, D), lambda i: (i, 0)),
            pl.BlockSpec((D, D), lambda i: (0, 0)),
        ],
        out_specs=pl.BlockSpec((MM_BLOCK, D), lambda i: (i, 0)),
        out_shape=jax.ShapeDtypeStruct((N_NODES, D), jnp.float32),
    )(p[0], p[1], w)


def _tc_sum(p):
    grid = (N_NODES // MM_BLOCK,)
    return pl.pallas_call(
        _add_body,
        grid=grid,
        in_specs=[
            pl.BlockSpec((MM_BLOCK, D), lambda i: (i, 0)),
            pl.BlockSpec((MM_BLOCK, D), lambda i: (i, 0)),
        ],
        out_specs=pl.BlockSpec((MM_BLOCK, D), lambda i: (i, 0)),
        out_shape=jax.ShapeDtypeStruct((N_NODES, D), jnp.float32),
    )(p[0], p[1])


def _make_sc_segsum(n_chunks):
    mesh = plsc.VectorSubcoreMesh(core_axis_name="c", subcore_axis_name="s")

    @functools.partial(
        pl.kernel,
        mesh=mesh,
        out_type=jax.ShapeDtypeStruct((NC, N_SP, D), jnp.float32),
        scratch_types=(
            [pltpu.VMEM((n_chunks // 4, 4 * CHUNK), jnp.int32)]
            + [pltpu.VMEM((CHUNK, D), jnp.float32) for _ in range(NBUF)]
            + [pltpu.VMEM((CHUNK,), jnp.int32) for _ in range(2 * NBUF)]
            + [pltpu.VMEM_SHARED((N_SP, D), jnp.float32)]
            + [pltpu.SemaphoreType.DMA for _ in range(2 * NBUF)]
        ),
    )
    def segsum(h_hbm, pidx_hbm, out_hbm, pidx_v, *rest):
        rows = rest[:NBUF]
        sidx = rest[NBUF:2 * NBUF]
        didx = rest[2 * NBUF:3 * NBUF]
        acc_sh = rest[3 * NBUF]
        gsem = rest[3 * NBUF + 1:4 * NBUF + 1]
        ssem = rest[4 * NBUF + 1:]
        c = lax.axis_index("c")
        s = lax.axis_index("s")
        wid = s * NC + c

        pltpu.sync_copy(pidx_hbm.at[wid], pidx_v)

        def unpack(row, col, b):
            for k in range(CHUNK // 16):
                v = pidx_v[row, pl.ds(col + k * 16, 16)]
                sidx[b][pl.ds(k * 16, 16)] = lax.bitwise_and(v, 0xFFFF)
                didx[b][pl.ds(k * 16, 16)] = lax.shift_right_logical(v, 16)

        def gst(b):
            pltpu.async_copy(h_hbm.at[sidx[b]], rows[b], gsem[b])

        def sst(b):
            pltpu.async_copy(rows[b], acc_sh.at[didx[b]], ssem[b], add=True)

        def gwait(b):
            pltpu.make_async_copy(h_hbm.at[sidx[b]], rows[b], gsem[b]).wait()

        def swait(b):
            pltpu.make_async_copy(h_hbm.at[sidx[b]], rows[b], ssem[b]).wait()

        n = n_chunks
        for j in range(NBUF - 1):
            unpack(j // 4, (j % 4) * CHUNK, j)
            gst(j)

        z = jnp.zeros((16,), jnp.float32)
        zb = rows[NBUF - 1]

        def _zero_row(i, _):
            for k in range(D // 16):
                zb[i, pl.ds(k * 16, 16)] = z
            return 0

        lax.fori_loop(0, CHUNK, _zero_row, 0)
        zbase = s * ZROWS
        nfull = ZROWS // CHUNK
        for j in range(nfull):
            pltpu.sync_copy(zb, acc_sh.at[pl.ds(zbase + j * CHUNK, CHUNK)])
        rem = ZROWS - nfull * CHUNK
        if rem:
            pltpu.sync_copy(zb.at[pl.ds(0, rem)],
                            acc_sh.at[pl.ds(zbase + nfull * CHUNK, rem)])
        plsc.subcore_barrier()

        unpack((NBUF - 1) // 4, ((NBUF - 1) % 4) * CHUNK, NBUF - 1)
        gst(NBUF - 1); gwait(0); sst(0)

        @pl.loop(NBUF, n, step=NBUF)
        def _grp(j0):
            h = j0 // 4
            for b in range(NBUF):
                swait(b)
                unpack(h + b // 4, (b % 4) * CHUNK, b)
                gst(b)
                gwait((b + 1) % NBUF)
                sst((b + 1) % NBUF)

        for b in range(1, NBUF):
            gwait(b)
            sst(b)
        for b in range(NBUF):
            swait(b)
        plsc.subcore_barrier()

        pltpu.sync_copy(acc_sh.at[pl.ds(s * ROWS_PER_TILE, ROWS_PER_TILE)],
                        out_hbm.at[c, pl.ds(s * ROWS_PER_TILE, ROWS_PER_TILE)])

    return segsum


def kernel(x, edge_index, W1, W2, W3):
    src = jnp.asarray(edge_index[0], jnp.int32)
    dst = jnp.asarray(edge_index[1], jnp.int32)
    n_edges = src.shape[0]
    per_xfer = NW * CHUNK
    n_chunks = -(-n_edges // per_xfer)
    n_chunks = -(-n_chunks // NBUF) * NBUF
    e_pad = n_chunks * per_xfer
    pad = e_pad - n_edges
    if pad:
        r = jnp.arange(pad, dtype=jnp.int32)
        src = jnp.concatenate([src, (r * 131) % N_NODES])
        dst = jnp.concatenate([dst, N_NODES + (r % (N_SP - N_NODES))])
    pidx3 = jnp.bitwise_or(src, jnp.left_shift(dst, 16)).reshape(
        NW, n_chunks // 4, 4 * CHUNK)

    segsum = _make_sc_segsum(n_chunks)

    h = _tc_matmul(x, W1)
    p = segsum(h, pidx3)
    h = _tc_sum_matmul(p, W2)
    p = segsum(h, pidx3)
    h = _tc_sum_matmul(p, W3)
    p = segsum(h, pidx3)
    return _tc_sum(p)


# speedup vs baseline: 4.5288x; 1.0226x over previous
import functools

import jax
import jax.numpy as jnp
from jax import lax
from jax.experimental import pallas as pl
from jax.experimental.pallas import tpu as pltpu
from jax.experimental.pallas import tpu_sc as plsc

N_NODES = 10000
D = 128
CHUNK = 32
NC, NS = 2, 16
NW = NC * NS
N_SP = 10112
ROWS_PER_TILE = N_SP // NS
ZROWS = ROWS_PER_TILE
NBUF = 8
MM_BLOCK = 5000


def _mm_body(x_ref, w_ref, o_ref):
    o_ref[...] = jnp.dot(x_ref[...], w_ref[...], preferred_element_type=jnp.float32)


def _summ_body(a_ref, b_ref, w_ref, o_ref):
    o_ref[...] = jnp.dot(a_ref[...] + b_ref[...], w_ref[...],
                         preferred_element_type=jnp.float32)


def _add_body(a_ref, b_ref, o_ref):
    o_ref[...] = a_ref[...] + b_ref[...]


def _tc_matmul(x, w):
    grid = (N_NODES // MM_BLOCK,)
    return pl.pallas_call(
        _mm_body,
        grid=grid,
        in_specs=[
            pl.BlockSpec((MM_BLOCK, D), lambda i: (i, 0)),
            pl.BlockSpec((D, D), lambda i: (0, 0)),
        ],
        out_specs=pl.BlockSpec((MM_BLOCK, D), lambda i: (i, 0)),
        out_shape=jax.ShapeDtypeStruct((N_NODES, D), jnp.float32),
    )(x, w)


def _tc_sum_matmul(p, w):
    grid = (N_NODES // MM_BLOCK,)
    return pl.pallas_call(
        _summ_body,
        grid=grid,
        in_specs=[
            pl.BlockSpec((MM_BLOCK, D), lambda i: (i, 0)),
            pl.BlockSpec((MM_BLOCK, D), lambda i: (i, 0)),
            pl.BlockSpec((D, D), lambda i: (0, 0)),
        ],
        out_specs=pl.BlockSpec((MM_BLOCK, D), lambda i: (i, 0)),
        out_shape=jax.ShapeDtypeStruct((N_NODES, D), jnp.float32),
    )(p[0], p[1], w)


def _tc_sum(p):
    grid = (N_NODES // MM_BLOCK,)
    return pl.pallas_call(
        _add_body,
        grid=grid,
        in_specs=[
            pl.BlockSpec((MM_BLOCK, D), lambda i: (i, 0)),
            pl.BlockSpec((MM_BLOCK, D), lambda i: (i, 0)),
        ],
        out_specs=pl.BlockSpec((MM_BLOCK, D), lambda i: (i, 0)),
        out_shape=jax.ShapeDtypeStruct((N_NODES, D), jnp.float32),
    )(p[0], p[1])


def _make_sc_segsum(n_chunks):
    mesh = plsc.VectorSubcoreMesh(core_axis_name="c", subcore_axis_name="s")

    @functools.partial(
        pl.kernel,
        mesh=mesh,
        out_type=jax.ShapeDtypeStruct((NC, N_SP, D), jnp.float32),
        scratch_types=(
            [pltpu.VMEM((n_chunks // 4, 4 * CHUNK), jnp.int32)]
            + [pltpu.VMEM((CHUNK, D), jnp.float32) for _ in range(NBUF)]
            + [pltpu.VMEM((CHUNK,), jnp.int32) for _ in range(2 * NBUF)]
            + [pltpu.VMEM_SHARED((N_SP, D), jnp.float32)]
            + [pltpu.SemaphoreType.DMA for _ in range(2 * NBUF)]
        ),
    )
    def segsum(h_hbm, pidx_hbm, out_hbm, pidx_v, *rest):
        rows = rest[:NBUF]
        sidx = rest[NBUF:2 * NBUF]
        didx = rest[2 * NBUF:3 * NBUF]
        acc_sh = rest[3 * NBUF]
        gsem = rest[3 * NBUF + 1:4 * NBUF + 1]
        ssem = rest[4 * NBUF + 1:]
        c = lax.axis_index("c")
        s = lax.axis_index("s")
        wid = s * NC + c

        pltpu.sync_copy(pidx_hbm.at[wid], pidx_v)

        def unpack(row, col, b):
            for k in range(CHUNK // 16):
                v = pidx_v[row, pl.ds(col + k * 16, 16)]
                sidx[b][pl.ds(k * 16, 16)] = lax.bitwise_and(v, 0xFFFF)
                didx[b][pl.ds(k * 16, 16)] = lax.shift_right_logical(v, 16)

        def gst(b):
            pltpu.async_copy(h_hbm.at[sidx[b]], rows[b], gsem[b])

        def sst(b):
            pltpu.async_copy(rows[b], acc_sh.at[didx[b]], ssem[b], add=True)

        def gwait(b):
            pltpu.make_async_copy(h_hbm.at[sidx[b]], rows[b], gsem[b]).wait()

        def swait(b):
            pltpu.make_async_copy(h_hbm.at[sidx[b]], rows[b], ssem[b]).wait()

        n = n_chunks
        for j in range(NBUF - 1):
            unpack(j // 4, (j % 4) * CHUNK, j)
            gst(j)

        z = jnp.zeros((16,), jnp.float32)
        zb = rows[NBUF - 1]

        def _zero_row(i, _):
            for k in range(D // 16):
                zb[i, pl.ds(k * 16, 16)] = z
            return 0

        lax.fori_loop(0, CHUNK, _zero_row, 0)
        zbase = s * ZROWS
        nfull = ZROWS // CHUNK
        for j in range(nfull):
            pltpu.sync_copy(zb, acc_sh.at[pl.ds(zbase + j * CHUNK, CHUNK)])
        rem = ZROWS - nfull * CHUNK
        if rem:
            pltpu.sync_copy(zb.at[pl.ds(0, rem)],
                            acc_sh.at[pl.ds(zbase + nfull * CHUNK, rem)])
        plsc.subcore_barrier()

        unpack((NBUF - 1) // 4, ((NBUF - 1) % 4) * CHUNK, NBUF - 1)
        gst(NBUF - 1); gwait(0); sst(0)

        @pl.loop(NBUF, n, step=NBUF)
        def _grp(j0):
            h = j0 // 4
            for b in range(NBUF):
                swait(b)
                unpack(h + b // 4, (b % 4) * CHUNK, b)
                gst(b)
                gwait((b + 1) % NBUF)
                sst((b + 1) % NBUF)

        for b in range(1, NBUF):
            gwait(b)
            sst(b)
        for b in range(NBUF):
            swait(b)
        plsc.subcore_barrier()

        pltpu.sync_copy(acc_sh.at[pl.ds(s * ROWS_PER_TILE, ROWS_PER_TILE)],
                        out_hbm.at[c, pl.ds(s * ROWS_PER_TILE, ROWS_PER_TILE)])

    return segsum


def kernel(x, edge_index, W1, W2, W3):
    src = jnp.asarray(edge_index[0], jnp.int32)
    dst = jnp.asarray(edge_index[1], jnp.int32)
    n_edges = src.shape[0]
    per_xfer = NW * CHUNK
    n_chunks = -(-n_edges // per_xfer)
    n_chunks = -(-n_chunks // NBUF) * NBUF
    e_pad = n_chunks * per_xfer
    pad = e_pad - n_edges
    if pad:
        r = jnp.arange(pad, dtype=jnp.int32)
        src = jnp.concatenate([src, (r * 131) % N_NODES])
        dst = jnp.concatenate([dst, N_NODES + (r % (N_SP - N_NODES))])
    pidx3 = jnp.bitwise_or(src, jnp.left_shift(dst, 16)).reshape(
        NW, n_chunks // 4, 4 * CHUNK)

    segsum = _make_sc_segsum(n_chunks)

    h = _tc_matmul(x, W1)
    p = segsum(h, pidx3)
    h = _tc_sum_matmul(p, W2)
    p = segsum(h, pidx3)
    h = _tc_sum_matmul(p, W3)
    p = segsum(h, pidx3)
    return _tc_sum(p)
